# baseline (speedup 1.0000x reference)
import functools

import jax
import jax.numpy as jnp
from jax import lax
from jax.experimental import pallas as pl
from jax.experimental.pallas import tpu as pltpu
from jax.experimental.pallas import tpu_sc as plsc

N = 10000
D = 128
K = 5000
NP = 10240
IT = 1024
KP = 5120
NW = 32
R_PER = KP // NW
CP = 5008
GRP = 8


def _scores_body(h_ref, w_ref, b_ref, out_ref, outr_ref):
    i = pl.program_id(0)
    w = jnp.dot(h_ref[...], w_ref[...], preferred_element_type=jnp.float32)
    s = jax.nn.sigmoid(w + b_ref[0, 0])
    row = i * IT + lax.broadcasted_iota(jnp.int32, (IT, 1), 0)
    s = jnp.where(row < N, s, -1.0)
    out_ref[...] = s
    outr_ref[...] = s.reshape(1, IT)


def _ranks_body(scol_ref, sflat_ref, out_ref):
    i = pl.program_id(0)
    j = pl.program_id(1)

    @pl.when(j == 0)
    def _():
        out_ref[...] = jnp.zeros_like(out_ref)

    s_i = scol_ref[...]
    s_j = sflat_ref[...]
    ig = i * IT + lax.broadcasted_iota(jnp.int32, (IT, 1), 0)
    jg = j * IT + lax.broadcasted_iota(jnp.int32, (1, IT), 1)
    m = (s_j > s_i).astype(jnp.float32) + ((s_j == s_i) & (jg < ig)).astype(jnp.float32)
    ones = jnp.ones((IT, 1), jnp.float32)
    out_ref[...] += jnp.dot(m, ones, preferred_element_type=jnp.float32).astype(jnp.int32)


def _select_body(rcol_ref, scol_ref, idx_ref, val_ref):
    p = pl.program_id(0)
    i = pl.program_id(1)

    @pl.when(i == 0)
    def _():
        idx_ref[...] = jnp.zeros_like(idx_ref)
        val_ref[...] = jnp.zeros_like(val_ref)

    r_i = rcol_ref[...]
    s_i = scol_ref[...]
    pg = p * IT + lax.broadcasted_iota(jnp.int32, (1, IT), 1)
    ig = i * IT + lax.broadcasted_iota(jnp.int32, (IT, 1), 0)
    eq = (r_i == pg).astype(jnp.float32)
    igf = ig.astype(jnp.float32).reshape(1, IT)
    idx_ref[...] += jnp.dot(igf, eq, preferred_element_type=jnp.float32).astype(jnp.int32)
    val_ref[...] += jnp.dot(s_i.reshape(1, IT), eq,
                            preferred_element_type=jnp.float32)


def _topk_idx_values(h, W, b):
    h_pad = jnp.pad(h, ((0, NP - N), (0, 0)))
    scores_col, scores_flat = pl.pallas_call(
        _scores_body,
        grid=(NP // IT,),
        in_specs=[
            pl.BlockSpec((IT, D), lambda i: (i, 0)),
            pl.BlockSpec((D, 1), lambda i: (0, 0)),
            pl.BlockSpec((1, 1), lambda i: (0, 0)),
        ],
        out_specs=[
            pl.BlockSpec((IT, 1), lambda i: (i, 0)),
            pl.BlockSpec((1, IT), lambda i: (0, i)),
        ],
        out_shape=[
            jax.ShapeDtypeStruct((NP, 1), jnp.float32),
            jax.ShapeDtypeStruct((1, NP), jnp.float32),
        ],
    )(h_pad, W, b.reshape(1, 1))

    ranks_col = pl.pallas_call(
        _ranks_body,
        grid=(NP // IT, NP // IT),
        in_specs=[
            pl.BlockSpec((IT, 1), lambda i, j: (i, 0)),
            pl.BlockSpec((1, IT), lambda i, j: (0, j)),
        ],
        out_specs=pl.BlockSpec((IT, 1), lambda i, j: (i, 0)),
        out_shape=jax.ShapeDtypeStruct((NP, 1), jnp.int32),
    )(scores_col, scores_flat)

    idx_row, val_row = pl.pallas_call(
        _select_body,
        grid=(KP // IT, NP // IT),
        in_specs=[
            pl.BlockSpec((IT, 1), lambda p, i: (i, 0)),
            pl.BlockSpec((IT, 1), lambda p, i: (i, 0)),
        ],
        out_specs=[
            pl.BlockSpec((1, IT), lambda p, i: (0, p)),
            pl.BlockSpec((1, IT), lambda p, i: (0, p)),
        ],
        out_shape=[
            jax.ShapeDtypeStruct((1, KP), jnp.int32),
            jax.ShapeDtypeStruct((1, KP), jnp.float32),
        ],
    )(ranks_col, scores_col)

    return idx_row.reshape(KP), val_row.reshape(KP)


def _sc_body(g_hbm, h_hbm, idx_hbm, vrep_hbm, ung_hbm, newh_hbm,
             idxv, rowbuf, outa, outb, hbuf, vrepv, sem, sema, semb):
    wid = lax.axis_index("s") * 2 + lax.axis_index("c")
    base = wid * R_PER
    count = jnp.minimum(R_PER, K - base)

    pltpu.sync_copy(idx_hbm, idxv)
    pltpu.sync_copy(vrep_hbm.at[pl.ds(base * 16, R_PER * 16)], vrepv)

    cps = [
        pltpu.async_copy(
            h_hbm.at[idxv.at[pl.ds(base + q * 80, 80)]],
            hbuf.at[pl.ds(q * 80, 80)], sem)
        for q in range(R_PER // 80)
    ]
    for cp in cps:
        cp.wait()

    def _scale_row(jj, carry):
        vb = vrepv[pl.ds(jj * 16, 16)]
        for c in range(D // 16):
            hv = hbuf[jj, pl.ds(c * 16, 16)]
            hbuf[jj, pl.ds(c * 16, 16)] = hv * vb
        return carry

    lax.fori_loop(0, count, _scale_row, 0, unroll=False)
    pltpu.sync_copy(hbuf, newh_hbm.at[pl.ds(base, R_PER)])

    def _group(q, carry):
        i0 = base + q * GRP
        pltpu.async_copy(g_hbm.at[idxv.at[pl.ds(i0, GRP)]], rowbuf, sem).wait()

        descs = []
        for jj in range(GRP):
            row_sel = jnp.full((16,), jj, jnp.int32)
            ob = outa if jj % 2 == 0 else outb
            osem = sema if jj % 2 == 0 else semb
            if jj >= 2:
                descs[jj - 2].wait()

            @plsc.parallel_loop(0, CP, step=16, unroll=4)
            def _chunks(c, ob=ob, row_sel=row_sel):
                ic = idxv[pl.ds(c, 16)]
                vals = plsc.load_gather(rowbuf, [row_sel, ic])
                ob[0, pl.ds(c, 16)] = (vals != 0.0).astype(jnp.float32)

            i = i0 + jj
            ci = (i // 16) * 16
            lane = jnp.full((16,), i % 16, jnp.int32)
            dsel = (lax.iota(jnp.int32, 16) == lane).astype(jnp.float32)
            ob[0, pl.ds(ci, 16)] = ob[0, pl.ds(ci, 16)] + dsel
            descs.append(pltpu.async_copy(ob.at[:, pl.ds(0, K)],
                                          ung_hbm.at[pl.ds(i, 1)], osem))
        descs[-2].wait()
        descs[-1].wait()
        return carry

    lax.fori_loop(0, count // GRP, _group, 0, unroll=False)


def _sc_gather(g, h, idx_pad, vrep):
    mesh = plsc.VectorSubcoreMesh(core_axis_name="c", subcore_axis_name="s")
    f = pl.kernel(
        _sc_body,
        out_type=[
            jax.ShapeDtypeStruct((K, K), jnp.float32),
            jax.ShapeDtypeStruct((KP, D), jnp.float32),
        ],
        mesh=mesh,
        compiler_params=pltpu.CompilerParams(
            use_tc_tiling_on_sc=False, needs_layout_passes=False),
        scratch_types=[
            pltpu.VMEM((KP,), jnp.int32),
            pltpu.VMEM((GRP, N), jnp.float32),
            pltpu.VMEM((1, CP), jnp.float32),
            pltpu.VMEM((1, CP), jnp.float32),
            pltpu.VMEM((R_PER, D), jnp.float32),
            pltpu.VMEM((R_PER * 16,), jnp.float32),
            pltpu.SemaphoreType.DMA,
            pltpu.SemaphoreType.DMA,
            pltpu.SemaphoreType.DMA,
        ],
    )
    return f(g, h, idx_pad, vrep)


def kernel(g, h, W, b):
    idx_pad, val_pad = _topk_idx_values(h, W, b)
    vrep = jnp.broadcast_to(val_pad[:, None], (KP, 16)).reshape(KP * 16)
    un_g, newh_pad = _sc_gather(g, h, idx_pad, vrep)
    return (un_g, newh_pad[:K], idx_pad[:K])


# speedup vs baseline: 1.3481x; 1.3481x over previous
import functools

import jax
import jax.numpy as jnp
from jax import lax
from jax.experimental import pallas as pl
from jax.experimental.pallas import tpu as pltpu
from jax.experimental.pallas import tpu_sc as plsc

N = 10000
D = 128
K = 5000
NP = 10240
IT = 1024
KP = 5120
NW = 32
R_PER = KP // NW
CP = 5008
GRP = 8


def _scores_body(h_ref, w_ref, b_ref, out_ref, outr_ref):
    i = pl.program_id(0)
    w = jnp.dot(h_ref[...], w_ref[...], preferred_element_type=jnp.float32)
    s = jax.nn.sigmoid(w + b_ref[0, 0])
    row = i * IT + lax.broadcasted_iota(jnp.int32, (IT, 1), 0)
    s = jnp.where(row < N, s, -1.0)
    out_ref[...] = s
    outr_ref[...] = s.reshape(1, IT)


def _ranks_body(scol_ref, sflat_ref, out_ref):
    i = pl.program_id(0)
    j = pl.program_id(1)

    @pl.when(j == 0)
    def _():
        out_ref[...] = jnp.zeros_like(out_ref)

    s_i = scol_ref[...]
    s_j = sflat_ref[...]
    ig = i * IT + lax.broadcasted_iota(jnp.int32, (IT, 1), 0)
    jg = j * IT + lax.broadcasted_iota(jnp.int32, (1, IT), 1)
    m = (s_j > s_i).astype(jnp.float32) + ((s_j == s_i) & (jg < ig)).astype(jnp.float32)
    ones = jnp.ones((IT, 1), jnp.float32)
    out_ref[...] += jnp.dot(m, ones, preferred_element_type=jnp.float32).astype(jnp.int32)


def _select_body(rcol_ref, scol_ref, idx_ref, val_ref):
    p = pl.program_id(0)
    i = pl.program_id(1)

    @pl.when(i == 0)
    def _():
        idx_ref[...] = jnp.zeros_like(idx_ref)
        val_ref[...] = jnp.zeros_like(val_ref)

    r_i = rcol_ref[...]
    s_i = scol_ref[...]
    pg = p * IT + lax.broadcasted_iota(jnp.int32, (1, IT), 1)
    ig = i * IT + lax.broadcasted_iota(jnp.int32, (IT, 1), 0)
    eq = (r_i == pg)
    idx_ref[...] += jnp.sum(jnp.where(eq, ig, 0), axis=0, keepdims=True)
    val_ref[...] += jnp.sum(jnp.where(eq, s_i, 0.0), axis=0, keepdims=True)


def _topk_idx_values(h, W, b):
    h_pad = jnp.pad(h, ((0, NP - N), (0, 0)))
    scores_col, scores_flat = pl.pallas_call(
        _scores_body,
        grid=(NP // IT,),
        in_specs=[
            pl.BlockSpec((IT, D), lambda i: (i, 0)),
            pl.BlockSpec((D, 1), lambda i: (0, 0)),
            pl.BlockSpec((1, 1), lambda i: (0, 0)),
        ],
        out_specs=[
            pl.BlockSpec((IT, 1), lambda i: (i, 0)),
            pl.BlockSpec((1, IT), lambda i: (0, i)),
        ],
        out_shape=[
            jax.ShapeDtypeStruct((NP, 1), jnp.float32),
            jax.ShapeDtypeStruct((1, NP), jnp.float32),
        ],
    )(h_pad, W, b.reshape(1, 1))

    ranks_col = pl.pallas_call(
        _ranks_body,
        grid=(NP // IT, NP // IT),
        in_specs=[
            pl.BlockSpec((IT, 1), lambda i, j: (i, 0)),
            pl.BlockSpec((1, IT), lambda i, j: (0, j)),
        ],
        out_specs=pl.BlockSpec((IT, 1), lambda i, j: (i, 0)),
        out_shape=jax.ShapeDtypeStruct((NP, 1), jnp.int32),
    )(scores_col, scores_flat)

    idx_row, val_row = pl.pallas_call(
        _select_body,
        grid=(KP // IT, NP // IT),
        in_specs=[
            pl.BlockSpec((IT, 1), lambda p, i: (i, 0)),
            pl.BlockSpec((IT, 1), lambda p, i: (i, 0)),
        ],
        out_specs=[
            pl.BlockSpec((1, IT), lambda p, i: (0, p)),
            pl.BlockSpec((1, IT), lambda p, i: (0, p)),
        ],
        out_shape=[
            jax.ShapeDtypeStruct((1, KP), jnp.int32),
            jax.ShapeDtypeStruct((1, KP), jnp.float32),
        ],
    )(ranks_col, scores_col)

    return idx_row.reshape(KP), val_row.reshape(KP)


def _sc_body(g_hbm, h_hbm, idx_hbm, vrep_hbm, ung_hbm, newh_hbm,
             idxv, rowbuf, outa, outb, hbuf, vrepv, sem, sema, semb):
    wid = lax.axis_index("s") * 2 + lax.axis_index("c")
    base = wid * R_PER
    count = jnp.minimum(R_PER, K - base)

    pltpu.sync_copy(idx_hbm, idxv)
    pltpu.sync_copy(vrep_hbm.at[pl.ds(base * 16, R_PER * 16)], vrepv)

    cps = [
        pltpu.async_copy(
            h_hbm.at[idxv.at[pl.ds(base + q * 80, 80)]],
            hbuf.at[pl.ds(q * 80, 80)], sem)
        for q in range(R_PER // 80)
    ]
    for cp in cps:
        cp.wait()

    def _scale_row(jj, carry):
        vb = vrepv[pl.ds(jj * 16, 16)]
        for c in range(D // 16):
            hv = hbuf[jj, pl.ds(c * 16, 16)]
            hbuf[jj, pl.ds(c * 16, 16)] = hv * vb
        return carry

    lax.fori_loop(0, count, _scale_row, 0, unroll=False)
    pltpu.sync_copy(hbuf, newh_hbm.at[pl.ds(base, R_PER)])

    def _group(q, carry):
        i0 = base + q * GRP
        pltpu.async_copy(g_hbm.at[idxv.at[pl.ds(i0, GRP)]], rowbuf, sem).wait()

        descs = []
        for jj in range(GRP):
            row_sel = jnp.full((16,), jj, jnp.int32)
            ob = outa if jj % 2 == 0 else outb
            osem = sema if jj % 2 == 0 else semb
            if jj >= 2:
                descs[jj - 2].wait()

            @plsc.parallel_loop(0, CP, step=16, unroll=4)
            def _chunks(c, ob=ob, row_sel=row_sel):
                ic = idxv[pl.ds(c, 16)]
                vals = plsc.load_gather(rowbuf, [row_sel, ic])
                ob[0, pl.ds(c, 16)] = (vals != 0.0).astype(jnp.float32)

            i = i0 + jj
            ci = (i // 16) * 16
            lane = jnp.full((16,), i % 16, jnp.int32)
            dsel = (lax.iota(jnp.int32, 16) == lane).astype(jnp.float32)
            ob[0, pl.ds(ci, 16)] = ob[0, pl.ds(ci, 16)] + dsel
            descs.append(pltpu.async_copy(ob.at[:, pl.ds(0, K)],
                                          ung_hbm.at[pl.ds(i, 1)], osem))
        descs[-2].wait()
        descs[-1].wait()
        return carry

    lax.fori_loop(0, count // GRP, _group, 0, unroll=False)


def _sc_gather(g, h, idx_pad, vrep):
    mesh = plsc.VectorSubcoreMesh(core_axis_name="c", subcore_axis_name="s")
    f = pl.kernel(
        _sc_body,
        out_type=[
            jax.ShapeDtypeStruct((K, K), jnp.float32),
            jax.ShapeDtypeStruct((KP, D), jnp.float32),
        ],
        mesh=mesh,
        compiler_params=pltpu.CompilerParams(
            use_tc_tiling_on_sc=False, needs_layout_passes=False),
        scratch_types=[
            pltpu.VMEM((KP,), jnp.int32),
            pltpu.VMEM((GRP, N), jnp.float32),
            pltpu.VMEM((1, CP), jnp.float32),
            pltpu.VMEM((1, CP), jnp.float32),
            pltpu.VMEM((R_PER, D), jnp.float32),
            pltpu.VMEM((R_PER * 16,), jnp.float32),
            pltpu.SemaphoreType.DMA,
            pltpu.SemaphoreType.DMA,
            pltpu.SemaphoreType.DMA,
        ],
    )
    return f(g, h, idx_pad, vrep)


def kernel(g, h, W, b):
    idx_pad, val_pad = _topk_idx_values(h, W, b)
    vrep = jnp.broadcast_to(val_pad[:, None], (KP, 16)).reshape(KP * 16)
    un_g, newh_pad = _sc_gather(g, h, idx_pad, vrep)
    return (un_g, newh_pad[:K], idx_pad[:K])
